# baseline (speedup 1.0000x reference)
import functools

import jax
import jax.numpy as jnp
from jax import lax
from jax.experimental import pallas as pl
from jax.experimental.pallas import tpu as pltpu
from jax.experimental.pallas import tpu_sc as plsc

C = 128
O = 128
OW = O // 2
WPT = 4
N = 10000
K = 16
L = 16

NC, NS = 2, 16
N_PAD = 10240
NSC = N_PAD // NC
CHN = 256
NCHU = NSC // CHN
BN = 2560
BN2 = 2560


def _mm_body(x_ref, wd_ref, w2_ref, hb_ref, t1_ref, t2_ref):
    xb = x_ref[...]
    hb = hb_ref[:, 0:1]
    dn = (((1,), (0,)), ((), ()))

    def pack(wmat):
        z = (lax.dot_general(wmat, xb, dn,
                             preferred_element_type=jnp.float32)
             + hb).astype(jnp.bfloat16)
        lo = lax.bitcast_convert_type(z[:OW, :], jnp.uint16).astype(jnp.uint32)
        hi = lax.bitcast_convert_type(z[OW:, :], jnp.uint16).astype(jnp.uint32)
        return lax.bitcast_convert_type(lo | (hi << 16), jnp.int32)

    t1_ref[...] = pack(wd_ref[...])
    t2_ref[...] = pack(w2_ref[...])


def _build_tables(xf, wd, w2, hb):
    return pl.pallas_call(
        _mm_body,
        out_shape=[
            jax.ShapeDtypeStruct((OW, N), jnp.int32),
            jax.ShapeDtypeStruct((OW, N), jnp.int32),
        ],
    )(xf, wd, w2, hb)


def _unpack_body(t_ref, o_ref):
    u = lax.bitcast_convert_type(t_ref[:, :N], jnp.uint32)
    lo = lax.bitcast_convert_type((u & 0xFFFF).astype(jnp.uint16),
                                  jnp.bfloat16).astype(jnp.float32)
    hi = lax.bitcast_convert_type((u >> 16).astype(jnp.uint16),
                                  jnp.bfloat16).astype(jnp.float32)
    o_ref[0:OW, :] = lo
    o_ref[OW:O, :] = hi


def _unpack(out_t):
    return pl.pallas_call(
        _unpack_body,
        out_shape=jax.ShapeDtypeStruct((O, N), jnp.float32),
    )(out_t)


@functools.partial(
    pl.kernel,
    mesh=plsc.VectorSubcoreMesh(core_axis_name="c", subcore_axis_name="s"),
    out_type=jax.ShapeDtypeStruct((OW, N_PAD), jnp.int32),
    compiler_params=pltpu.CompilerParams(use_tc_tiling_on_sc=False,
                                         needs_layout_passes=False),
    scratch_types=[
        pltpu.VMEM((WPT, N), jnp.int32),
        pltpu.VMEM((WPT, N), jnp.int32),
        pltpu.VMEM((2, K, CHN), jnp.int32),
        pltpu.VMEM((2, K, CHN), jnp.int32),
        pltpu.VMEM((2, WPT, CHN), jnp.int32),
        pltpu.SemaphoreType.DMA,
        pltpu.SemaphoreType.DMA,
        pltpu.SemaphoreType.DMA,
        pltpu.SemaphoreType.DMA,
    ],
)
def _sc_gather_max(y1t_hbm, y2t_hbm, i1_hbm, i0_hbm, out_hbm,
                   tbl1, tbl2, i1_v, i0_v, o_v, si0, si1, so0, so1):
    cid = lax.axis_index("c")
    sid = lax.axis_index("s")
    sis = (si0, si1)
    sos = (so0, so1)
    wrows = pl.ds(sid * WPT, WPT)

    pltpu.sync_copy(y1t_hbm.at[wrows], tbl1)
    pltpu.sync_copy(y2t_hbm.at[wrows], tbl2)

    wvecs = [jnp.full((L,), w, jnp.int32) for w in range(WPT)]

    def issue_idx(ch, b):
        pltpu.async_copy(i1_hbm.at[cid, ch], i1_v.at[b], sis[b])
        pltpu.async_copy(i0_hbm.at[cid, ch], i0_v.at[b], sis[b])

    def wait_idx(ch, b):
        pltpu.make_async_copy(i1_hbm.at[cid, ch], i1_v.at[b], sis[b]).wait()
        pltpu.make_async_copy(i0_hbm.at[cid, ch], i0_v.at[b], sis[b]).wait()

    def out_slice(ch):
        return out_hbm.at[wrows, pl.ds(cid * NSC + ch * CHN, CHN)]

    def compute(b):
        def group_body(g, carry):
            sl = pl.ds(g * L, L)
            acc = [None] * WPT
            for k in range(K):
                idx1 = i1_v[b, k, sl]
                idx0 = i0_v[b, k, sl]
                for w in range(WPT):
                    s = (plsc.bitcast(plsc.load_gather(tbl1, [wvecs[w], idx1]),
                                      jnp.bfloat16)
                         + plsc.bitcast(plsc.load_gather(tbl2, [wvecs[w], idx0]),
                                        jnp.bfloat16))
                    acc[w] = s if k == 0 else jnp.maximum(acc[w], s)
            for w in range(WPT):
                o_v[b, w, sl] = plsc.bitcast(acc[w], jnp.int32)
            return carry

        lax.fori_loop(0, CHN // L, group_body, 0, unroll=False)

    issue_idx(0, 0)
    issue_idx(1, 1)

    def pair_body(cp, carry):
        for b in range(2):
            ch = 2 * cp + b
            wait_idx(ch, b)

            @pl.when(cp > 0)
            def _():
                pltpu.make_async_copy(o_v.at[b], out_slice(ch - 2), sos[b]).wait()

            compute(b)
            pltpu.async_copy(o_v.at[b], out_slice(ch), sos[b])

            @pl.when(cp < NCHU // 2 - 1)
            def _():
                issue_idx(ch + 2, b)
        return carry

    lax.fori_loop(0, NCHU // 2, pair_body, 0, unroll=False)
    pltpu.make_async_copy(o_v.at[0], out_slice(NCHU - 2), sos[0]).wait()
    pltpu.make_async_copy(o_v.at[1], out_slice(NCHU - 1), sos[1]).wait()


def kernel(x, edge_index, W, b):
    wd = (W[:, :C] - W[:, C:]).astype(jnp.float32)
    w2 = W[:, C:].astype(jnp.float32)
    hb = jnp.broadcast_to(0.5 * b.astype(jnp.float32)[:, None], (O, 128))

    ei = edge_index.astype(jnp.int32)

    def prep_idx(a):
        ap = jnp.pad(a, ((0, N_PAD - N), (0, 0)))
        return ap.T.reshape(K, NC, NCHU, CHN).transpose(1, 2, 0, 3)

    i1 = prep_idx(ei[1, 0])
    i0 = prep_idx(ei[0, 0])

    t1, t2 = _build_tables(x.reshape(C, N).astype(jnp.float32), wd, w2, hb)
    out_t = _sc_gather_max(t1, t2, i1, i0)
    return _unpack(out_t).reshape(1, O, N, 1)


# speedup vs baseline: 1.0059x; 1.0059x over previous
import functools

import jax
import jax.numpy as jnp
from jax import lax
from jax.experimental import pallas as pl
from jax.experimental.pallas import tpu as pltpu
from jax.experimental.pallas import tpu_sc as plsc

C = 128
O = 128
OW = O // 2
WPT = 4
N = 10000
K = 16
L = 16

NC, NS = 2, 16
N_PAD = 10240
NSC = N_PAD // NC
CHN = 256
NCHU = NSC // CHN
BN = 2560
BN2 = 2560


def _mm_body(x_ref, wd_ref, w2_ref, hb_ref, t1_ref, t2_ref):
    xb = x_ref[...]
    hb = hb_ref[:, 0:1]
    dn = (((1,), (0,)), ((), ()))

    def pack(wmat):
        z = (lax.dot_general(wmat, xb, dn,
                             preferred_element_type=jnp.float32)
             + hb).astype(jnp.bfloat16)
        lo = lax.bitcast_convert_type(z[:OW, :], jnp.uint16).astype(jnp.uint32)
        hi = lax.bitcast_convert_type(z[OW:, :], jnp.uint16).astype(jnp.uint32)
        return lax.bitcast_convert_type(lo | (hi << 16), jnp.int32)

    t1_ref[...] = pack(wd_ref[...])
    t2_ref[...] = pack(w2_ref[...])


def _build_tables(xp, wd, w2, hb):
    return pl.pallas_call(
        _mm_body,
        grid=(N_PAD // BN,),
        in_specs=[
            pl.BlockSpec((C, BN), lambda i: (0, i)),
            pl.BlockSpec((O, C), lambda i: (0, 0)),
            pl.BlockSpec((O, C), lambda i: (0, 0)),
            pl.BlockSpec((O, 128), lambda i: (0, 0)),
        ],
        out_specs=[
            pl.BlockSpec((OW, BN), lambda i: (0, i)),
            pl.BlockSpec((OW, BN), lambda i: (0, i)),
        ],
        out_shape=[
            jax.ShapeDtypeStruct((OW, N_PAD), jnp.int32),
            jax.ShapeDtypeStruct((OW, N_PAD), jnp.int32),
        ],
    )(xp, wd, w2, hb)


def _unpack_body(t_ref, o_ref):
    u = lax.bitcast_convert_type(t_ref[...], jnp.uint32)
    lo = lax.bitcast_convert_type((u & 0xFFFF).astype(jnp.uint16),
                                  jnp.bfloat16).astype(jnp.float32)
    hi = lax.bitcast_convert_type((u >> 16).astype(jnp.uint16),
                                  jnp.bfloat16).astype(jnp.float32)
    o_ref[0:OW, :] = lo
    o_ref[OW:O, :] = hi


def _unpack(out_t):
    return pl.pallas_call(
        _unpack_body,
        grid=(N_PAD // BN2,),
        in_specs=[pl.BlockSpec((OW, BN2), lambda i: (0, i))],
        out_specs=pl.BlockSpec((O, BN2), lambda i: (0, i)),
        out_shape=jax.ShapeDtypeStruct((O, N_PAD), jnp.float32),
    )(out_t)


@functools.partial(
    pl.kernel,
    mesh=plsc.VectorSubcoreMesh(core_axis_name="c", subcore_axis_name="s"),
    out_type=jax.ShapeDtypeStruct((OW, N_PAD), jnp.int32),
    compiler_params=pltpu.CompilerParams(use_tc_tiling_on_sc=False,
                                         needs_layout_passes=False),
    scratch_types=[
        pltpu.VMEM((WPT, N_PAD), jnp.int32),
        pltpu.VMEM((WPT, N_PAD), jnp.int32),
        pltpu.VMEM((2, K, CHN), jnp.int32),
        pltpu.VMEM((2, K, CHN), jnp.int32),
        pltpu.VMEM((2, WPT, CHN), jnp.int32),
        pltpu.SemaphoreType.DMA,
        pltpu.SemaphoreType.DMA,
        pltpu.SemaphoreType.DMA,
        pltpu.SemaphoreType.DMA,
    ],
)
def _sc_gather_max(y1t_hbm, y2t_hbm, i1_hbm, i0_hbm, out_hbm,
                   tbl1, tbl2, i1_v, i0_v, o_v, si0, si1, so0, so1):
    cid = lax.axis_index("c")
    sid = lax.axis_index("s")
    sis = (si0, si1)
    sos = (so0, so1)
    wrows = pl.ds(sid * WPT, WPT)

    pltpu.sync_copy(y1t_hbm.at[wrows], tbl1)
    pltpu.sync_copy(y2t_hbm.at[wrows], tbl2)

    wvecs = [jnp.full((L,), w, jnp.int32) for w in range(WPT)]

    def issue_idx(ch, b):
        pltpu.async_copy(i1_hbm.at[cid, ch], i1_v.at[b], sis[b])
        pltpu.async_copy(i0_hbm.at[cid, ch], i0_v.at[b], sis[b])

    def wait_idx(ch, b):
        pltpu.make_async_copy(i1_hbm.at[cid, ch], i1_v.at[b], sis[b]).wait()
        pltpu.make_async_copy(i0_hbm.at[cid, ch], i0_v.at[b], sis[b]).wait()

    def out_slice(ch):
        return out_hbm.at[wrows, pl.ds(cid * NSC + ch * CHN, CHN)]

    def compute(b):
        def group_body(g, carry):
            sl = pl.ds(g * L, L)
            acc = [None] * WPT
            for k in range(K):
                idx1 = i1_v[b, k, sl]
                idx0 = i0_v[b, k, sl]
                for w in range(WPT):
                    s = (plsc.bitcast(plsc.load_gather(tbl1, [wvecs[w], idx1]),
                                      jnp.bfloat16)
                         + plsc.bitcast(plsc.load_gather(tbl2, [wvecs[w], idx0]),
                                        jnp.bfloat16))
                    acc[w] = s if k == 0 else jnp.maximum(acc[w], s)
            for w in range(WPT):
                o_v[b, w, sl] = plsc.bitcast(acc[w], jnp.int32)
            return carry

        lax.fori_loop(0, CHN // L, group_body, 0, unroll=False)

    issue_idx(0, 0)
    issue_idx(1, 1)

    def pair_body(cp, carry):
        for b in range(2):
            ch = 2 * cp + b
            wait_idx(ch, b)

            @pl.when(cp > 0)
            def _():
                pltpu.make_async_copy(o_v.at[b], out_slice(ch - 2), sos[b]).wait()

            compute(b)
            pltpu.async_copy(o_v.at[b], out_slice(ch), sos[b])

            @pl.when(cp < NCHU // 2 - 1)
            def _():
                issue_idx(ch + 2, b)
        return carry

    lax.fori_loop(0, NCHU // 2, pair_body, 0, unroll=False)
    pltpu.make_async_copy(o_v.at[0], out_slice(NCHU - 2), sos[0]).wait()
    pltpu.make_async_copy(o_v.at[1], out_slice(NCHU - 1), sos[1]).wait()


def kernel(x, edge_index, W, b):
    wd = (W[:, :C] - W[:, C:]).astype(jnp.float32)
    w2 = W[:, C:].astype(jnp.float32)
    hb = jnp.broadcast_to(0.5 * b.astype(jnp.float32)[:, None], (O, 128))

    ei = edge_index.astype(jnp.int32)

    def prep_idx(a):
        ap = jnp.pad(a, ((0, N_PAD - N), (0, 0)))
        return ap.T.reshape(K, NC, NCHU, CHN).transpose(1, 2, 0, 3)

    i1 = prep_idx(ei[1, 0])
    i0 = prep_idx(ei[0, 0])

    xp = jnp.pad(x.reshape(C, N).astype(jnp.float32), ((0, 0), (0, N_PAD - N)))
    t1, t2 = _build_tables(xp, wd, w2, hb)
    out_t = _sc_gather_max(t1, t2, i1, i0)
    return _unpack(out_t)[:, :N].reshape(1, O, N, 1)
